# baseline (speedup 1.0000x reference)
import functools

import jax
import jax.numpy as jnp
from jax import lax
from jax.experimental import pallas as pl
from jax.experimental.pallas import tpu as pltpu
from jax.experimental.pallas import tpu_sc as plsc

N = 10000
E = 320000
D = 128
DH = D // 2

NC = 2
NS = 16
NW = NC * NS

EPW = E // NW
K = 125
NCHUNK = EPW // K
KD = 80
NCHUNKD = EPW // KD
RING = 8
LOOKAHEAD = 4
ZW = 1000
NZB = N // ZW
CH = 50

_mesh = plsc.VectorSubcoreMesh(core_axis_name="c", subcore_axis_name="s")


@functools.partial(
    pl.kernel,
    out_type=jax.ShapeDtypeStruct((NC, NZB, 1, ZW), jnp.float32),
    mesh=_mesh,
    scratch_types=[
        pltpu.VMEM((NCHUNKD, KD), jnp.int32),
        pltpu.VMEM((KD,), jnp.float32),
        pltpu.VMEM((ZW,), jnp.float32),
        pltpu.VMEM_SHARED((N,), jnp.float32),
        pltpu.SemaphoreType.DMA,
    ],
    compiler_params=pltpu.CompilerParams(use_tc_tiling_on_sc=False),
)
def _deg_partials(dst_hbm, degp_hbm, dst_v, ones_v, zb_v, deg_sh, dsem):
    c = lax.axis_index("c")
    s = lax.axis_index("s")
    w = c * NS + s
    zeros16 = jnp.zeros((16,), jnp.float32)
    ones16 = jnp.ones((16,), jnp.float32)

    def fill_ones(i, _):
        ones_v[pl.ds(i * 16, 16)] = ones16
        return 0

    lax.fori_loop(0, KD // 16, fill_ones, 0)

    def fill_zb(i, _):
        zb_v[pl.ds(i * 16, 16)] = zeros16
        return 0

    lax.fori_loop(0, ZW // 16, fill_zb, 0)

    @pl.when(s < NZB)
    def _():
        pltpu.sync_copy(zb_v, deg_sh.at[pl.ds(s * ZW, ZW)])

    pltpu.sync_copy(dst_hbm.at[w], dst_v)
    plsc.subcore_barrier()

    def count_body(j, _):
        pltpu.async_copy(ones_v, deg_sh.at[dst_v.at[j]], dsem, add=True)
        return 0

    lax.fori_loop(0, NCHUNKD, count_body, 0)

    def count_drain(j, _):
        pltpu.make_async_copy(ones_v, deg_sh.at[dst_v.at[0]], dsem).wait()
        return 0

    lax.fori_loop(0, NCHUNKD, count_drain, 0)
    plsc.subcore_barrier()

    @pl.when(s < NZB)
    def _():
        pltpu.sync_copy(deg_sh.at[pl.ds(s * ZW, ZW)], zb_v)
        pltpu.sync_copy(zb_v, degp_hbm.at[c, s, 0])


@functools.partial(
    pl.kernel,
    out_type=jax.ShapeDtypeStruct((NC, 2, N, DH), jnp.float32),
    mesh=_mesh,
    scratch_types=[
        pltpu.VMEM((NCHUNK, K), jnp.int32),
        pltpu.VMEM((NCHUNK, K), jnp.int32),
        pltpu.VMEM((RING, K, DH), jnp.float32),
        pltpu.VMEM((CH, DH), jnp.float32),
        pltpu.VMEM((CH, DH), jnp.float32),
        pltpu.VMEM_SHARED((N, DH), jnp.float32),
        pltpu.SemaphoreType.DMA((RING,)),
        pltpu.SemaphoreType.DMA((RING,)),
    ],
    compiler_params=pltpu.CompilerParams(use_tc_tiling_on_sc=False),
)
def _edge_segsum(yl_hbm, yh_hbm, src_hbm, dst_hbm, out_hbm,
                 src_v, dst_v, rows_v, zbuf, wbuf, acc_sh, gsem, ssem):
    c = lax.axis_index("c")
    s = lax.axis_index("s")
    w = c * NS + s
    zeros16 = jnp.zeros((16,), jnp.float32)

    def zbuf_body(i, _):
        zbuf[i // (DH // 16), pl.ds((i % (DH // 16)) * 16, 16)] = zeros16
        return 0

    lax.fori_loop(0, CH * (DH // 16), zbuf_body, 0)
    pltpu.sync_copy(src_hbm.at[w], src_v)
    pltpu.sync_copy(dst_hbm.at[w], dst_v)

    for h, y_hbm in enumerate((yl_hbm, yh_hbm)):
        @pl.when(s < NZB)
        def _():
            def zero_acc(t, _):
                pltpu.sync_copy(zbuf, acc_sh.at[pl.ds(s * ZW + t * CH, CH)])
                return 0

            lax.fori_loop(0, ZW // CH, zero_acc, 0)

        plsc.subcore_barrier()

        for p in range(LOOKAHEAD):
            pltpu.async_copy(y_hbm.at[src_v.at[p]], rows_v.at[p],
                             gsem.at[p])

        def edge_block(blk, _):
            for p in range(RING):
                j = RING * blk + p
                pltpu.make_async_copy(y_hbm.at[src_v.at[j]],
                                      rows_v.at[p], gsem.at[p]).wait()
                pltpu.async_copy(rows_v.at[p], acc_sh.at[dst_v.at[j]],
                                 ssem.at[p], add=True)
                q = (p + LOOKAHEAD) % RING
                jn = j + LOOKAHEAD

                @pl.when(jn < NCHUNK)
                def _(p=p, q=q, j=j, jn=jn):
                    @pl.when(jn >= RING)
                    def _():
                        pltpu.make_async_copy(
                            rows_v.at[q],
                            acc_sh.at[dst_v.at[0]],
                            ssem.at[q]).wait()

                    pltpu.async_copy(y_hbm.at[src_v.at[jn]], rows_v.at[q],
                                     gsem.at[q])
            return 0

        lax.fori_loop(0, NCHUNK // RING, edge_block, 0)
        for j in range(NCHUNK - RING, NCHUNK):
            b = j % RING
            pltpu.make_async_copy(rows_v.at[b], acc_sh.at[dst_v.at[0]],
                                  ssem.at[b]).wait()
        plsc.subcore_barrier()

        @pl.when(s < NZB)
        def _():
            def write_body(t, _):
                r0 = s * ZW + t * CH
                pltpu.sync_copy(acc_sh.at[pl.ds(r0, CH)], wbuf)
                pltpu.sync_copy(wbuf, out_hbm.at[c, h, pl.ds(r0, CH)])
                return 0

            lax.fori_loop(0, ZW // CH, write_body, 0)

        plsc.subcore_barrier()


BR = 1000


def _dinv_block(degp_blk):
    deg = 1.0 + degp_blk[0, 0, 0] + degp_blk[1, 0, 0]
    return lax.rsqrt(deg)[:, None]


def _acc_block(acc_blk):
    return jnp.concatenate(
        [acc_blk[0, 0] + acc_blk[1, 0], acc_blk[0, 1] + acc_blk[1, 1]],
        axis=1)


def _split_out(y, ys_ref):
    ys_ref[0] = y[:, :DH]
    ys_ref[1] = y[:, DH:]


def _y1_body(x_ref, w_ref, degp_ref, y_ref, ys_ref):
    dinv = _dinv_block(degp_ref[...])
    y = jnp.dot(x_ref[...], w_ref[...],
                preferred_element_type=jnp.float32) * dinv
    y_ref[...] = y
    _split_out(y, ys_ref)


def _y2_body(acc_ref, y1_ref, w_ref, degp_ref, y2_ref, ys_ref):
    dinv = _dinv_block(degp_ref[...])
    h = jax.nn.relu((_acc_block(acc_ref[...]) + y1_ref[...]) * dinv)
    y2 = jnp.dot(h, w_ref[...], preferred_element_type=jnp.float32) * dinv
    y2_ref[...] = y2
    _split_out(y2, ys_ref)


def _out_body(acc_ref, y2_ref, degp_ref, o_ref):
    dinv = _dinv_block(degp_ref[...])
    o_ref[...] = (_acc_block(acc_ref[...]) + y2_ref[...]) * dinv


_row_spec = pl.BlockSpec((BR, D), lambda i: (i, 0))
_w_spec = pl.BlockSpec((D, D), lambda i: (0, 0))
_degp_spec = pl.BlockSpec((NC, 1, 1, ZW), lambda i: (0, i, 0, 0))
_acc_spec = pl.BlockSpec((NC, 2, BR, DH), lambda i: (0, 0, i, 0))
_ys_spec = pl.BlockSpec((2, BR, DH), lambda i: (0, i, 0))
_grid = (N // BR,)
_row_out = jax.ShapeDtypeStruct((N, D), jnp.float32)
_ys_out = jax.ShapeDtypeStruct((2, N, DH), jnp.float32)

_y1_call = pl.pallas_call(
    _y1_body, grid=_grid,
    in_specs=[_row_spec, _w_spec, _degp_spec],
    out_specs=[_row_spec, _ys_spec], out_shape=[_row_out, _ys_out])

_y2_call = pl.pallas_call(
    _y2_body, grid=_grid,
    in_specs=[_acc_spec, _row_spec, _w_spec, _degp_spec],
    out_specs=[_row_spec, _ys_spec], out_shape=[_row_out, _ys_out])

_out_call = pl.pallas_call(
    _out_body, grid=_grid,
    in_specs=[_acc_spec, _row_spec, _degp_spec],
    out_specs=_row_spec, out_shape=_row_out)


def kernel(x, edge_index, W1, W2):
    src = edge_index[0].reshape(NW, NCHUNK, K)
    dst = edge_index[1].reshape(NW, NCHUNK, K)
    dst_d = edge_index[1].reshape(NW, NCHUNKD, KD)

    degp = _deg_partials(dst_d)
    y1, y1s = _y1_call(x, W1, degp)
    acc1 = _edge_segsum(y1s[0], y1s[1], src, dst)
    y2, y2s = _y2_call(acc1, y1, W2, degp)
    acc2 = _edge_segsum(y2s[0], y2s[1], src, dst)
    return _out_call(acc2, y2, degp)


# speedup vs baseline: 1.0325x; 1.0325x over previous
import functools

import jax
import jax.numpy as jnp
from jax import lax
from jax.experimental import pallas as pl
from jax.experimental.pallas import tpu as pltpu
from jax.experimental.pallas import tpu_sc as plsc

N = 10000
E = 320000
D = 128
DH = D // 2

NC = 2
NS = 16
NW = NC * NS

EPW = E // NW
K = 125
NCHUNK = EPW // K
KD = 80
NCHUNKD = EPW // KD
RING = 5
LOOKAHEAD = 3
ZW = 1000
NZB = N // ZW
CH = 200

_mesh = plsc.VectorSubcoreMesh(core_axis_name="c", subcore_axis_name="s")


@functools.partial(
    pl.kernel,
    out_type=jax.ShapeDtypeStruct((NC, NZB, 1, ZW), jnp.float32),
    mesh=_mesh,
    scratch_types=[
        pltpu.VMEM((NCHUNKD, KD), jnp.int32),
        pltpu.VMEM((KD,), jnp.float32),
        pltpu.VMEM((ZW,), jnp.float32),
        pltpu.VMEM_SHARED((N,), jnp.float32),
        pltpu.SemaphoreType.DMA,
    ],
    compiler_params=pltpu.CompilerParams(use_tc_tiling_on_sc=False),
)
def _deg_partials(dst_hbm, degp_hbm, dst_v, ones_v, zb_v, deg_sh, dsem):
    c = lax.axis_index("c")
    s = lax.axis_index("s")
    w = c * NS + s
    zeros16 = jnp.zeros((16,), jnp.float32)
    ones16 = jnp.ones((16,), jnp.float32)

    def fill_ones(i, _):
        ones_v[pl.ds(i * 16, 16)] = ones16
        return 0

    lax.fori_loop(0, KD // 16, fill_ones, 0)

    def fill_zb(i, _):
        zb_v[pl.ds(i * 16, 16)] = zeros16
        return 0

    lax.fori_loop(0, ZW // 16, fill_zb, 0)

    @pl.when(s < NZB)
    def _():
        pltpu.sync_copy(zb_v, deg_sh.at[pl.ds(s * ZW, ZW)])

    pltpu.sync_copy(dst_hbm.at[w], dst_v)
    plsc.subcore_barrier()

    def count_body(j, _):
        pltpu.sync_copy(ones_v, deg_sh.at[dst_v.at[j]], add=True)
        return 0

    lax.fori_loop(0, NCHUNKD, count_body, 0)
    plsc.subcore_barrier()

    @pl.when(s < NZB)
    def _():
        pltpu.sync_copy(deg_sh.at[pl.ds(s * ZW, ZW)], zb_v)
        pltpu.sync_copy(zb_v, degp_hbm.at[c, s, 0])


@functools.partial(
    pl.kernel,
    out_type=jax.ShapeDtypeStruct((NC, 2, N, DH), jnp.float32),
    mesh=_mesh,
    scratch_types=[
        pltpu.VMEM((NCHUNK, K), jnp.int32),
        pltpu.VMEM((NCHUNK, K), jnp.int32),
        pltpu.VMEM((RING, K, DH), jnp.float32),
        pltpu.VMEM((CH, DH), jnp.float32),
        pltpu.VMEM((CH, DH), jnp.float32),
        pltpu.VMEM_SHARED((N, DH), jnp.float32),
        pltpu.SemaphoreType.DMA((RING,)),
        pltpu.SemaphoreType.DMA((RING,)),
    ],
    compiler_params=pltpu.CompilerParams(use_tc_tiling_on_sc=False),
)
def _edge_segsum(yl_hbm, yh_hbm, src_hbm, dst_hbm, out_hbm,
                 src_v, dst_v, rows_v, zbuf, wbuf, acc_sh, gsem, ssem):
    c = lax.axis_index("c")
    s = lax.axis_index("s")
    w = c * NS + s
    zeros16 = jnp.zeros((16,), jnp.float32)

    def zbuf_body(i, _):
        zbuf[i // (DH // 16), pl.ds((i % (DH // 16)) * 16, 16)] = zeros16
        return 0

    lax.fori_loop(0, CH * (DH // 16), zbuf_body, 0)
    pltpu.sync_copy(src_hbm.at[w], src_v)
    pltpu.sync_copy(dst_hbm.at[w], dst_v)

    for h, y_hbm in enumerate((yl_hbm, yh_hbm)):
        @pl.when(s < NZB)
        def _():
            def zero_acc(t, _):
                pltpu.sync_copy(zbuf, acc_sh.at[pl.ds(s * ZW + t * CH, CH)])
                return 0

            lax.fori_loop(0, ZW // CH, zero_acc, 0)

        plsc.subcore_barrier()

        for p in range(LOOKAHEAD):
            pltpu.async_copy(y_hbm.at[src_v.at[p]], rows_v.at[p],
                             gsem.at[p])

        def edge_block(blk, _):
            for p in range(RING):
                j = RING * blk + p
                pltpu.make_async_copy(y_hbm.at[src_v.at[j]],
                                      rows_v.at[p], gsem.at[p]).wait()
                pltpu.async_copy(rows_v.at[p], acc_sh.at[dst_v.at[j]],
                                 ssem.at[p], add=True)
                q = (p + LOOKAHEAD) % RING
                jn = j + LOOKAHEAD

                @pl.when(jn < NCHUNK)
                def _(p=p, q=q, j=j, jn=jn):
                    @pl.when(jn >= RING)
                    def _():
                        pltpu.make_async_copy(
                            rows_v.at[q],
                            acc_sh.at[dst_v.at[0]],
                            ssem.at[q]).wait()

                    pltpu.async_copy(y_hbm.at[src_v.at[jn]], rows_v.at[q],
                                     gsem.at[q])
            return 0

        lax.fori_loop(0, NCHUNK // RING, edge_block, 0)
        for j in range(NCHUNK - RING, NCHUNK):
            b = j % RING
            pltpu.make_async_copy(rows_v.at[b], acc_sh.at[dst_v.at[0]],
                                  ssem.at[b]).wait()
        plsc.subcore_barrier()

        @pl.when(s < NZB)
        def _():
            def write_body(t, _):
                r0 = s * ZW + t * CH
                pltpu.sync_copy(acc_sh.at[pl.ds(r0, CH)], wbuf)
                pltpu.sync_copy(wbuf, out_hbm.at[c, h, pl.ds(r0, CH)])
                return 0

            lax.fori_loop(0, ZW // CH, write_body, 0)

        plsc.subcore_barrier()


BR = 1000


def _dinv_block(degp_blk):
    deg = 1.0 + degp_blk[0, 0, 0] + degp_blk[1, 0, 0]
    return lax.rsqrt(deg)[:, None]


def _acc_block(acc_blk):
    return jnp.concatenate(
        [acc_blk[0, 0] + acc_blk[1, 0], acc_blk[0, 1] + acc_blk[1, 1]],
        axis=1)


def _split_out(y, ys_ref):
    ys_ref[0] = y[:, :DH]
    ys_ref[1] = y[:, DH:]


def _y1_body(x_ref, w_ref, degp_ref, y_ref, ys_ref):
    dinv = _dinv_block(degp_ref[...])
    y = jnp.dot(x_ref[...], w_ref[...],
                preferred_element_type=jnp.float32) * dinv
    y_ref[...] = y
    _split_out(y, ys_ref)


def _y2_body(acc_ref, y1_ref, w_ref, degp_ref, y2_ref, ys_ref):
    dinv = _dinv_block(degp_ref[...])
    h = jax.nn.relu((_acc_block(acc_ref[...]) + y1_ref[...]) * dinv)
    y2 = jnp.dot(h, w_ref[...], preferred_element_type=jnp.float32) * dinv
    y2_ref[...] = y2
    _split_out(y2, ys_ref)


def _out_body(acc_ref, y2_ref, degp_ref, o_ref):
    dinv = _dinv_block(degp_ref[...])
    o_ref[...] = (_acc_block(acc_ref[...]) + y2_ref[...]) * dinv


_row_spec = pl.BlockSpec((BR, D), lambda i: (i, 0))
_w_spec = pl.BlockSpec((D, D), lambda i: (0, 0))
_degp_spec = pl.BlockSpec((NC, 1, 1, ZW), lambda i: (0, i, 0, 0))
_acc_spec = pl.BlockSpec((NC, 2, BR, DH), lambda i: (0, 0, i, 0))
_ys_spec = pl.BlockSpec((2, BR, DH), lambda i: (0, i, 0))
_grid = (N // BR,)
_row_out = jax.ShapeDtypeStruct((N, D), jnp.float32)
_ys_out = jax.ShapeDtypeStruct((2, N, DH), jnp.float32)

_y1_call = pl.pallas_call(
    _y1_body, grid=_grid,
    in_specs=[_row_spec, _w_spec, _degp_spec],
    out_specs=[_row_spec, _ys_spec], out_shape=[_row_out, _ys_out])

_y2_call = pl.pallas_call(
    _y2_body, grid=_grid,
    in_specs=[_acc_spec, _row_spec, _w_spec, _degp_spec],
    out_specs=[_row_spec, _ys_spec], out_shape=[_row_out, _ys_out])

_out_call = pl.pallas_call(
    _out_body, grid=_grid,
    in_specs=[_acc_spec, _row_spec, _degp_spec],
    out_specs=_row_spec, out_shape=_row_out)


def kernel(x, edge_index, W1, W2):
    src = edge_index[0].reshape(NW, NCHUNK, K)
    dst = edge_index[1].reshape(NW, NCHUNK, K)
    dst_d = edge_index[1].reshape(NW, NCHUNKD, KD)

    degp = _deg_partials(dst_d)
    y1, y1s = _y1_call(x, W1, degp)
    acc1 = _edge_segsum(y1s[0], y1s[1], src, dst)
    y2, y2s = _y2_call(acc1, y1, W2, degp)
    acc2 = _edge_segsum(y2s[0], y2s[1], src, dst)
    return _out_call(acc2, y2, degp)


# speedup vs baseline: 1.0572x; 1.0240x over previous
import functools

import jax
import jax.numpy as jnp
from jax import lax
from jax.experimental import pallas as pl
from jax.experimental.pallas import tpu as pltpu
from jax.experimental.pallas import tpu_sc as plsc

N = 10000
E = 320000
D = 128
DH = D // 2

NC = 2
NS = 16
NW = NC * NS

EPW = E // NW
K = 125
NCHUNK = EPW // K
RING = 5
LOOKAHEAD = 3
ZW = 1000
NZB = N // ZW
CH = 200

_mesh = plsc.VectorSubcoreMesh(core_axis_name="c", subcore_axis_name="s")


@functools.partial(
    pl.kernel,
    out_type=jax.ShapeDtypeStruct((NC, NZB, 1, ZW), jnp.float32),
    mesh=_mesh,
    scratch_types=[
        pltpu.VMEM((NCHUNK, K), jnp.int32),
        pltpu.VMEM((128,), jnp.float32),
        pltpu.VMEM((ZW,), jnp.float32),
        pltpu.VMEM_SHARED((N,), jnp.float32),
    ],
    compiler_params=pltpu.CompilerParams(use_tc_tiling_on_sc=False),
)
def _deg_partials(ei_hbm, degp_hbm, dst_v, ones_v, zb_v, deg_sh):
    c = lax.axis_index("c")
    s = lax.axis_index("s")
    w = c * NS + s
    zeros16 = jnp.zeros((16,), jnp.float32)
    ones16 = jnp.ones((16,), jnp.float32)

    def fill_ones(i, _):
        ones_v[pl.ds(i * 16, 16)] = ones16
        return 0

    lax.fori_loop(0, 128 // 16, fill_ones, 0)

    def fill_zb(i, _):
        zb_v[pl.ds(i * 16, 16)] = zeros16
        return 0

    lax.fori_loop(0, ZW // 16, fill_zb, 0)

    @pl.when(s < NZB)
    def _():
        pltpu.sync_copy(zb_v, deg_sh.at[pl.ds(s * ZW, ZW)])

    pltpu.sync_copy(ei_hbm.at[1, w], dst_v)
    plsc.subcore_barrier()

    def count_body(j, _):
        pltpu.sync_copy(ones_v.at[pl.ds(0, K)], deg_sh.at[dst_v.at[j]],
                        add=True)
        return 0

    lax.fori_loop(0, NCHUNK, count_body, 0)
    plsc.subcore_barrier()

    @pl.when(s < NZB)
    def _():
        pltpu.sync_copy(deg_sh.at[pl.ds(s * ZW, ZW)], zb_v)
        pltpu.sync_copy(zb_v, degp_hbm.at[c, s, 0])


@functools.partial(
    pl.kernel,
    out_type=jax.ShapeDtypeStruct((NC, 2, N, DH), jnp.float32),
    mesh=_mesh,
    scratch_types=[
        pltpu.VMEM((NCHUNK, K), jnp.int32),
        pltpu.VMEM((NCHUNK, K), jnp.int32),
        pltpu.VMEM((RING, K, DH), jnp.float32),
        pltpu.VMEM((CH, DH), jnp.float32),
        pltpu.VMEM((CH, DH), jnp.float32),
        pltpu.VMEM_SHARED((N, DH), jnp.float32),
        pltpu.SemaphoreType.DMA((RING,)),
        pltpu.SemaphoreType.DMA((RING,)),
    ],
    compiler_params=pltpu.CompilerParams(use_tc_tiling_on_sc=False),
)
def _edge_segsum(yl_hbm, yh_hbm, ei_hbm, out_hbm,
                 src_v, dst_v, rows_v, zbuf, wbuf, acc_sh, gsem, ssem):
    c = lax.axis_index("c")
    s = lax.axis_index("s")
    w = c * NS + s
    zeros16 = jnp.zeros((16,), jnp.float32)

    def zbuf_body(i, _):
        zbuf[i // (DH // 16), pl.ds((i % (DH // 16)) * 16, 16)] = zeros16
        return 0

    lax.fori_loop(0, CH * (DH // 16), zbuf_body, 0)
    pltpu.sync_copy(ei_hbm.at[0, w], src_v)
    pltpu.sync_copy(ei_hbm.at[1, w], dst_v)

    for h, y_hbm in enumerate((yl_hbm, yh_hbm)):
        @pl.when(s < NZB)
        def _():
            def zero_acc(t, _):
                pltpu.sync_copy(zbuf, acc_sh.at[pl.ds(s * ZW + t * CH, CH)])
                return 0

            lax.fori_loop(0, ZW // CH, zero_acc, 0)

        plsc.subcore_barrier()

        for p in range(LOOKAHEAD):
            pltpu.async_copy(y_hbm.at[src_v.at[p]], rows_v.at[p],
                             gsem.at[p])

        def edge_block(blk, _):
            for p in range(RING):
                j = RING * blk + p
                pltpu.make_async_copy(y_hbm.at[src_v.at[j]],
                                      rows_v.at[p], gsem.at[p]).wait()
                pltpu.async_copy(rows_v.at[p], acc_sh.at[dst_v.at[j]],
                                 ssem.at[p], add=True)
                q = (p + LOOKAHEAD) % RING
                jn = j + LOOKAHEAD

                @pl.when(jn < NCHUNK)
                def _(p=p, q=q, j=j, jn=jn):
                    @pl.when(jn >= RING)
                    def _():
                        pltpu.make_async_copy(
                            rows_v.at[q],
                            acc_sh.at[dst_v.at[0]],
                            ssem.at[q]).wait()

                    pltpu.async_copy(y_hbm.at[src_v.at[jn]], rows_v.at[q],
                                     gsem.at[q])
            return 0

        lax.fori_loop(0, NCHUNK // RING, edge_block, 0)
        for j in range(NCHUNK - RING, NCHUNK):
            b = j % RING
            pltpu.make_async_copy(rows_v.at[b], acc_sh.at[dst_v.at[0]],
                                  ssem.at[b]).wait()
        plsc.subcore_barrier()

        @pl.when(s < NZB)
        def _():
            def write_body(t, _):
                r0 = s * ZW + t * CH
                pltpu.sync_copy(acc_sh.at[pl.ds(r0, CH)], wbuf)
                pltpu.sync_copy(wbuf, out_hbm.at[c, h, pl.ds(r0, CH)])
                return 0

            lax.fori_loop(0, ZW // CH, write_body, 0)

        plsc.subcore_barrier()


BR = 1000


def _dinv_block(degp_blk):
    deg = 1.0 + degp_blk[0, 0, 0] + degp_blk[1, 0, 0]
    return lax.rsqrt(deg)[:, None]


def _acc_block(acc_blk):
    return jnp.concatenate(
        [acc_blk[0, 0] + acc_blk[1, 0], acc_blk[0, 1] + acc_blk[1, 1]],
        axis=1)


def _split_out(y, ys_ref):
    ys_ref[0] = y[:, :DH]
    ys_ref[1] = y[:, DH:]


def _y1_body(x_ref, w_ref, degp_ref, y_ref, ys_ref):
    dinv = _dinv_block(degp_ref[...])
    y = jnp.dot(x_ref[...], w_ref[...],
                preferred_element_type=jnp.float32) * dinv
    y_ref[...] = y
    _split_out(y, ys_ref)


def _y2_body(acc_ref, y1_ref, w_ref, degp_ref, y2_ref, ys_ref):
    dinv = _dinv_block(degp_ref[...])
    h = jax.nn.relu((_acc_block(acc_ref[...]) + y1_ref[...]) * dinv)
    y2 = jnp.dot(h, w_ref[...], preferred_element_type=jnp.float32) * dinv
    y2_ref[...] = y2
    _split_out(y2, ys_ref)


def _out_body(acc_ref, y2_ref, degp_ref, o_ref):
    dinv = _dinv_block(degp_ref[...])
    o_ref[...] = (_acc_block(acc_ref[...]) + y2_ref[...]) * dinv


_row_spec = pl.BlockSpec((BR, D), lambda i: (i, 0))
_w_spec = pl.BlockSpec((D, D), lambda i: (0, 0))
_degp_spec = pl.BlockSpec((NC, 1, 1, ZW), lambda i: (0, i, 0, 0))
_acc_spec = pl.BlockSpec((NC, 2, BR, DH), lambda i: (0, 0, i, 0))
_ys_spec = pl.BlockSpec((2, BR, DH), lambda i: (0, i, 0))
_grid = (N // BR,)
_row_out = jax.ShapeDtypeStruct((N, D), jnp.float32)
_ys_out = jax.ShapeDtypeStruct((2, N, DH), jnp.float32)

_y1_call = pl.pallas_call(
    _y1_body, grid=_grid,
    in_specs=[_row_spec, _w_spec, _degp_spec],
    out_specs=[_row_spec, _ys_spec], out_shape=[_row_out, _ys_out])

_y2_call = pl.pallas_call(
    _y2_body, grid=_grid,
    in_specs=[_acc_spec, _row_spec, _w_spec, _degp_spec],
    out_specs=[_row_spec, _ys_spec], out_shape=[_row_out, _ys_out])

_out_call = pl.pallas_call(
    _out_body, grid=_grid,
    in_specs=[_acc_spec, _row_spec, _degp_spec],
    out_specs=_row_spec, out_shape=_row_out)


def kernel(x, edge_index, W1, W2):
    ei = edge_index.reshape(2, NW, NCHUNK, K)

    degp = _deg_partials(ei)
    y1, y1s = _y1_call(x, W1, degp)
    acc1 = _edge_segsum(y1s[0], y1s[1], ei)
    y2, y2s = _y2_call(acc1, y1, W2, degp)
    acc2 = _edge_segsum(y2s[0], y2s[1], ei)
    return _out_call(acc2, y2, degp)
